# baseline (speedup 1.0000x reference)
import functools

import jax
import jax.numpy as jnp
from jax import lax
from jax.experimental import pallas as pl
from jax.experimental.pallas import tpu as pltpu
from jax.experimental.pallas import tpu_sc as plsc

NUM_ITEMS = 1000000
DIM = 64
BATCH = 16384
MAX_NORM = 10.0

_B = 3 * BATCH
_CHUNK = 128
_NCHUNKS = _B // _CHUNK
_NW = 32
_CPW = _NCHUNKS // _NW
_HALF = _CPW // 2


def _sc_gather(t2, unc, idxh3d, idx3d):
    mesh = plsc.VectorSubcoreMesh(core_axis_name="c", subcore_axis_name="s")

    @functools.partial(
        pl.kernel,
        mesh=mesh,
        compiler_params=pltpu.CompilerParams(use_tc_tiling_on_sc=True),
        out_type=[
            jax.ShapeDtypeStruct((_B, 2 * DIM), jnp.float32),
            jax.ShapeDtypeStruct((_NW, _CPW, _CHUNK), jnp.float32),
        ],
        scratch_types=[
            pltpu.VMEM((_CPW, _CHUNK), jnp.int32),
            pltpu.VMEM((_CPW, _CHUNK), jnp.int32),
            pltpu.VMEM((_HALF * _CHUNK, 2 * DIM), jnp.float32),
            pltpu.VMEM((_CPW, _CHUNK), jnp.float32),
            pltpu.SemaphoreType.DMA,
            pltpu.SemaphoreType.DMA,
        ],
    )
    def gather_kernel(t2_h, unc_h, idxh_h, idx_h, rows_o, unc_o,
                      idxh_v, idx_v, rows_v, unc_v, sem_r, sem_u):
        wid = lax.axis_index("s") * 2 + lax.axis_index("c")
        pltpu.sync_copy(idxh_h.at[wid], idxh_v)
        pltpu.sync_copy(idx_h.at[wid], idx_v)
        unc_descs = []
        for c in range(_CPW):
            d = pltpu.make_async_copy(
                unc_h.at[idx_v.at[c]], unc_v.at[c], sem_u)
            d.start()
            unc_descs.append(d)
        for h in range(2):
            descs = []
            for c in range(_HALF):
                d = pltpu.make_async_copy(
                    t2_h.at[idxh_v.at[h * _HALF + c]],
                    rows_v.at[pl.ds(c * _CHUNK, _CHUNK)], sem_r)
                d.start()
                descs.append(d)
            for d in descs:
                d.wait()
            pltpu.sync_copy(
                rows_v,
                rows_o.at[pl.ds((wid * _CPW + h * _HALF) * _CHUNK,
                                _HALF * _CHUNK)])
        for d in unc_descs:
            d.wait()
        pltpu.sync_copy(unc_v, unc_o.at[wid])

    return gather_kernel(t2, unc, idxh3d, idx3d)


_SUM_LANES = 16384
_SUM_STEPS = -(-NUM_ITEMS // _SUM_LANES)
_SUM_VALID_LAST = NUM_ITEMS - (_SUM_STEPS - 1) * _SUM_LANES


def _tc_table_sum(tt):
    def body(t_ref, o_ref):
        i = pl.program_id(0)

        @pl.when(i == 0)
        def _():
            o_ref[...] = jnp.zeros_like(o_ref)

        valid = jnp.where(i == _SUM_STEPS - 1, _SUM_VALID_LAST, _SUM_LANES)
        lane = lax.broadcasted_iota(jnp.int32, (DIM, _SUM_LANES), 1)
        blk = jnp.where(lane < valid, t_ref[...], 0.0)
        ones = jnp.ones((_SUM_LANES, 1), jnp.float32)
        part = lax.dot_general(blk, ones, (((1,), (0,)), ((), ())),
                               preferred_element_type=jnp.float32)
        o_ref[...] += part

    return pl.pallas_call(
        body,
        grid=(_SUM_STEPS,),
        in_specs=[pl.BlockSpec((DIM, _SUM_LANES), lambda i: (0, i))],
        out_specs=pl.BlockSpec((DIM, 1), lambda i: (0, 0)),
        out_shape=jax.ShapeDtypeStruct((DIM, 1), jnp.float32),
    )(tt)


def _erfc(x):
    t = 1.0 / (1.0 + 0.5 * x)
    poly = 0.17087277
    for c in (-0.82215223, 1.48851587, -1.13520398, 0.27886807, -0.18628806,
              0.09678418, 0.37409196, 1.00002368, -1.26551223):
        poly = c + t * poly
    return t * jnp.exp(-x * x + poly)


_PAIR_STEPS = 8
_PG = (BATCH // 128) // _PAIR_STEPS


def _pair_body(rows_ref, par_ref, unc_ref, y_ref, tsum_ref, o_ref):
    def sumsq(x):
        return jnp.sum(x * x, axis=-1)

    def pick(k):
        row2 = rows_ref[k]
        par = par_ref[k][..., None]
        return jnp.where(par > 0.5, row2[..., DIM:], row2[..., :DIM])

    j = pick(0)
    e1 = pick(1)
    e2 = pick(2)

    def scale(r):
        n = jnp.sqrt(sumsq(r))
        return jnp.minimum(1.0, MAX_NORM / (n + 1e-7))[..., None]

    jn = j * scale(j)
    e1n = e1 * scale(e1)
    e2n = e2 * scale(e2)
    d1 = jnp.sqrt(sumsq(jn - e1n + 1e-6))
    d2 = jnp.sqrt(sumsq(jn - e2n + 1e-6))

    jv = jnp.exp(unc_ref[0]) + 1e-8
    v1 = jnp.exp(unc_ref[1]) + 1e-8
    v2 = jnp.exp(unc_ref[2]) + 1e-8
    s1v = jnp.sqrt(jv + v1 + 1e-8)
    s2v = jnp.sqrt(jv + v2 + 1e-8)

    p_hat = 1.0 / (1.0 + jnp.exp(d1 - d2))
    sigma = jnp.sqrt(p_hat * (1.0 - p_hat) * jnp.sqrt(s1v + s2v + 1e-8))
    z = p_hat / sigma
    w = z * 0.7071067811865476
    p = 0.5 * (2.0 - _erfc(w))
    p = jnp.clip(p, 1e-8, 1.0 - 1e-8)
    y = y_ref[...]
    bce = -(y * jnp.log(p) + (1.0 - y) * jnp.log(1.0 - p))

    i = pl.program_id(0)

    @pl.when(i == 0)
    def _():
        o_ref[0, 0] = 0.0

    o_ref[0, 0] += jnp.sum(bce)

    @pl.when(i == _PAIR_STEPS - 1)
    def _():
        o_ref[0, 0] = (o_ref[0, 0] * (1.0 / BATCH)
                       + jnp.sum(tsum_ref[...]) * (1e-6 / (NUM_ITEMS * DIM)))


def _tc_pair(rows4, par3, unc3, y2d, tsum):
    return pl.pallas_call(
        _pair_body,
        grid=(_PAIR_STEPS,),
        in_specs=[
            pl.BlockSpec((3, _PG, 128, 2 * DIM), lambda i: (0, i, 0, 0)),
            pl.BlockSpec((3, _PG, 128), lambda i: (0, i, 0)),
            pl.BlockSpec((3, _PG, 128), lambda i: (0, i, 0)),
            pl.BlockSpec((_PG, 128), lambda i: (i, 0)),
            pl.BlockSpec((DIM, 1), lambda i: (0, 0)),
        ],
        out_specs=pl.BlockSpec((1, 1), lambda i: (0, 0), memory_space=pltpu.SMEM),
        out_shape=jax.ShapeDtypeStruct((1, 1), jnp.float32),
    )(rows4, par3, unc3, y2d, tsum)


def kernel(table, uncertainties, pairs, comparisons):
    t2 = table.reshape(500000, 2 * DIM)
    idx = pairs.astype(jnp.int32).T
    idx3d = idx.reshape(_NW, _CPW, _CHUNK)
    idxh3d = (idx >> 1).reshape(_NW, _CPW, _CHUNK)
    par3 = (idx & 1).astype(jnp.float32).reshape(3, BATCH // 128, 128)
    rows, unc_g = _sc_gather(t2, uncertainties, idxh3d, idx3d)
    tsum = _tc_table_sum(table.T)
    loss = _tc_pair(
        rows.reshape(3, BATCH // 128, 128, 2 * DIM),
        par3,
        unc_g.reshape(3, BATCH // 128, 128),
        comparisons.reshape(BATCH // 128, 128),
        tsum,
    )
    return loss.reshape(())


# speedup vs baseline: 2.3713x; 2.3713x over previous
import functools

import jax
import jax.numpy as jnp
from jax import lax
from jax.experimental import pallas as pl
from jax.experimental.pallas import tpu as pltpu
from jax.experimental.pallas import tpu_sc as plsc

NUM_ITEMS = 1000000
DIM = 64
BATCH = 16384
MAX_NORM = 10.0

_B = 3 * BATCH
_CHUNK = 128
_NCHUNKS = _B // _CHUNK
_NW = 32
_CPW = _NCHUNKS // _NW
_HALF = _CPW // 2


def _sc_gather(t2, unc, idxh3d, idx3d):
    mesh = plsc.VectorSubcoreMesh(core_axis_name="c", subcore_axis_name="s")

    @functools.partial(
        pl.kernel,
        mesh=mesh,
        compiler_params=pltpu.CompilerParams(use_tc_tiling_on_sc=True),
        out_type=[
            jax.ShapeDtypeStruct((_B, 2 * DIM), jnp.float32),
            jax.ShapeDtypeStruct((_NW, _CPW, _CHUNK), jnp.float32),
        ],
        scratch_types=[
            pltpu.VMEM((_CPW, _CHUNK), jnp.int32),
            pltpu.VMEM((_CPW, _CHUNK), jnp.int32),
            pltpu.VMEM((_HALF * _CHUNK, 2 * DIM), jnp.float32),
            pltpu.VMEM((_CPW, _CHUNK), jnp.float32),
            pltpu.SemaphoreType.DMA,
            pltpu.SemaphoreType.DMA,
        ],
    )
    def gather_kernel(t2_h, unc_h, idxh_h, idx_h, rows_o, unc_o,
                      idxh_v, idx_v, rows_v, unc_v, sem_r, sem_u):
        wid = lax.axis_index("s") * 2 + lax.axis_index("c")
        pltpu.sync_copy(idxh_h.at[wid], idxh_v)
        pltpu.sync_copy(idx_h.at[wid], idx_v)
        unc_descs = []
        for c in range(_CPW):
            d = pltpu.make_async_copy(
                unc_h.at[idx_v.at[c]], unc_v.at[c], sem_u)
            d.start()
            unc_descs.append(d)
        for h in range(2):
            descs = []
            for c in range(_HALF):
                d = pltpu.make_async_copy(
                    t2_h.at[idxh_v.at[h * _HALF + c]],
                    rows_v.at[pl.ds(c * _CHUNK, _CHUNK)], sem_r)
                d.start()
                descs.append(d)
            for d in descs:
                d.wait()
            pltpu.sync_copy(
                rows_v,
                rows_o.at[pl.ds((wid * _CPW + h * _HALF) * _CHUNK,
                                _HALF * _CHUNK)])
        for d in unc_descs:
            d.wait()
        pltpu.sync_copy(unc_v, unc_o.at[wid])

    return gather_kernel(t2, unc, idxh3d, idx3d)


_SUM_LANES = 16384
_SUM_STEPS = -(-NUM_ITEMS // _SUM_LANES)
_SUM_VALID_LAST = NUM_ITEMS - (_SUM_STEPS - 1) * _SUM_LANES


def _tc_sum_and_detile(tt):
    def body(t_ref, o_sum, o_rm):
        i = pl.program_id(0)

        @pl.when(i == 0)
        def _():
            o_sum[...] = jnp.zeros_like(o_sum)

        x = t_ref[...]
        valid = jnp.where(i == _SUM_STEPS - 1, _SUM_VALID_LAST, _SUM_LANES)
        lane = lax.broadcasted_iota(jnp.int32, (DIM, _SUM_LANES), 1)
        blk = jnp.where(lane < valid, x, 0.0)
        ones = jnp.ones((_SUM_LANES, 1), jnp.float32)
        part = lax.dot_general(blk, ones, (((1,), (0,)), ((), ())),
                               preferred_element_type=jnp.float32)
        o_sum[...] += part

        o_rm[0] = jnp.concatenate(
            [x[:, :_SUM_LANES // 2].T, x[:, _SUM_LANES // 2:].T], axis=1)

    return pl.pallas_call(
        body,
        grid=(_SUM_STEPS,),
        in_specs=[pl.BlockSpec((DIM, _SUM_LANES), lambda i: (0, i))],
        out_specs=[
            pl.BlockSpec((DIM, 1), lambda i: (0, 0)),
            pl.BlockSpec((1, _SUM_LANES // 2, 2 * DIM), lambda i: (i, 0, 0)),
        ],
        out_shape=[
            jax.ShapeDtypeStruct((DIM, 1), jnp.float32),
            jax.ShapeDtypeStruct((_SUM_STEPS, _SUM_LANES // 2, 2 * DIM),
                                 jnp.float32),
        ],
    )(tt)


def _erfc(x):
    t = 1.0 / (1.0 + 0.5 * x)
    poly = 0.17087277
    for c in (-0.82215223, 1.48851587, -1.13520398, 0.27886807, -0.18628806,
              0.09678418, 0.37409196, 1.00002368, -1.26551223):
        poly = c + t * poly
    return t * jnp.exp(-x * x + poly)


_PAIR_STEPS = 8
_PG = (BATCH // 128) // _PAIR_STEPS


def _pair_body(rows_ref, par_ref, unc_ref, y_ref, tsum_ref, o_ref):
    def sumsq(x):
        return jnp.sum(x * x, axis=-1)

    def pick(k):
        row2 = rows_ref[k]
        par = par_ref[k][..., None]
        return jnp.where(par > 0.5, row2[..., DIM:], row2[..., :DIM])

    j = pick(0)
    e1 = pick(1)
    e2 = pick(2)

    def scale(r):
        n = jnp.sqrt(sumsq(r))
        return jnp.minimum(1.0, MAX_NORM / (n + 1e-7))[..., None]

    jn = j * scale(j)
    e1n = e1 * scale(e1)
    e2n = e2 * scale(e2)
    d1 = jnp.sqrt(sumsq(jn - e1n + 1e-6))
    d2 = jnp.sqrt(sumsq(jn - e2n + 1e-6))

    jv = jnp.exp(unc_ref[0]) + 1e-8
    v1 = jnp.exp(unc_ref[1]) + 1e-8
    v2 = jnp.exp(unc_ref[2]) + 1e-8
    s1v = jnp.sqrt(jv + v1 + 1e-8)
    s2v = jnp.sqrt(jv + v2 + 1e-8)

    p_hat = 1.0 / (1.0 + jnp.exp(d1 - d2))
    sigma = jnp.sqrt(p_hat * (1.0 - p_hat) * jnp.sqrt(s1v + s2v + 1e-8))
    z = p_hat / sigma
    w = z * 0.7071067811865476
    p = 0.5 * (2.0 - _erfc(w))
    p = jnp.clip(p, 1e-8, 1.0 - 1e-8)
    y = y_ref[...]
    bce = -(y * jnp.log(p) + (1.0 - y) * jnp.log(1.0 - p))

    i = pl.program_id(0)

    @pl.when(i == 0)
    def _():
        o_ref[0, 0] = 0.0

    o_ref[0, 0] += jnp.sum(bce)

    @pl.when(i == _PAIR_STEPS - 1)
    def _():
        o_ref[0, 0] = (o_ref[0, 0] * (1.0 / BATCH)
                       + jnp.sum(tsum_ref[...]) * (1e-6 / (NUM_ITEMS * DIM)))


def _tc_pair(rows4, par3, unc3, y2d, tsum):
    return pl.pallas_call(
        _pair_body,
        grid=(_PAIR_STEPS,),
        in_specs=[
            pl.BlockSpec((3, _PG, 128, 2 * DIM), lambda i: (0, i, 0, 0)),
            pl.BlockSpec((3, _PG, 128), lambda i: (0, i, 0)),
            pl.BlockSpec((3, _PG, 128), lambda i: (0, i, 0)),
            pl.BlockSpec((_PG, 128), lambda i: (i, 0)),
            pl.BlockSpec((DIM, 1), lambda i: (0, 0)),
        ],
        out_specs=pl.BlockSpec((1, 1), lambda i: (0, 0), memory_space=pltpu.SMEM),
        out_shape=jax.ShapeDtypeStruct((1, 1), jnp.float32),
    )(rows4, par3, unc3, y2d, tsum)


def kernel(table, uncertainties, pairs, comparisons):
    idx = pairs.astype(jnp.int32).T
    idx3d = idx.reshape(_NW, _CPW, _CHUNK)
    idxh3d = ((idx >> 14) * (_SUM_LANES // 2)
              + (idx & (_SUM_LANES // 2 - 1))).reshape(_NW, _CPW, _CHUNK)
    par3 = ((idx >> 13) & 1).astype(jnp.float32).reshape(3, BATCH // 128, 128)
    tsum, t_rm = _tc_sum_and_detile(table.T)
    t2 = t_rm.reshape(_SUM_STEPS * (_SUM_LANES // 2), 2 * DIM)
    rows, unc_g = _sc_gather(t2, uncertainties, idxh3d, idx3d)
    loss = _tc_pair(
        rows.reshape(3, BATCH // 128, 128, 2 * DIM),
        par3,
        unc_g.reshape(3, BATCH // 128, 128),
        comparisons.reshape(BATCH // 128, 128),
        tsum,
    )
    return loss.reshape(())


# speedup vs baseline: 2.4563x; 1.0358x over previous
import functools

import jax
import jax.numpy as jnp
from jax import lax
from jax.experimental import pallas as pl
from jax.experimental.pallas import tpu as pltpu
from jax.experimental.pallas import tpu_sc as plsc

NUM_ITEMS = 1000000
DIM = 64
BATCH = 16384
MAX_NORM = 10.0

_B = 3 * BATCH
_CHUNK = 128
_NCHUNKS = _B // _CHUNK
_NW = 32
_CPW = _NCHUNKS // _NW
_HALF = _CPW // 2


def _sc_gather(t64, unc, idxr3d, idx3d):
    mesh = plsc.VectorSubcoreMesh(core_axis_name="c", subcore_axis_name="s")

    @functools.partial(
        pl.kernel,
        mesh=mesh,
        compiler_params=pltpu.CompilerParams(use_tc_tiling_on_sc=False),
        out_type=[
            jax.ShapeDtypeStruct((_B, DIM), jnp.float32),
            jax.ShapeDtypeStruct((_NW, _CPW, _CHUNK), jnp.float32),
        ],
        scratch_types=[
            pltpu.VMEM((_CPW, _CHUNK), jnp.int32),
            pltpu.VMEM((_CPW, _CHUNK), jnp.int32),
            pltpu.VMEM((_CPW * _CHUNK, DIM), jnp.float32),
            pltpu.VMEM((_CPW, _CHUNK), jnp.float32),
            pltpu.SemaphoreType.DMA,
            pltpu.SemaphoreType.DMA,
        ],
    )
    def gather_kernel(t64_h, unc_h, idxr_h, idx_h, rows_o, unc_o,
                      idxr_v, idx_v, rows_v, unc_v, sem_r, sem_u):
        wid = lax.axis_index("s") * 2 + lax.axis_index("c")
        pltpu.sync_copy(idxr_h.at[wid], idxr_v)
        pltpu.sync_copy(idx_h.at[wid], idx_v)
        descs = []
        for c in range(_CPW):
            d = pltpu.make_async_copy(
                unc_h.at[idx_v.at[c]], unc_v.at[c], sem_u)
            d.start()
            descs.append(d)
        for c in range(_CPW):
            d = pltpu.make_async_copy(
                t64_h.at[idxr_v.at[c]],
                rows_v.at[pl.ds(c * _CHUNK, _CHUNK)], sem_r)
            d.start()
            descs.append(d)
        for d in descs:
            d.wait()
        pltpu.sync_copy(rows_v,
                        rows_o.at[pl.ds(wid * _CPW * _CHUNK, _CPW * _CHUNK)])
        pltpu.sync_copy(unc_v, unc_o.at[wid])

    return gather_kernel(t64, unc, idxr3d, idx3d)


_SUM_LANES = 16384
_SUM_STEPS = -(-NUM_ITEMS // _SUM_LANES)
_SUM_VALID_LAST = NUM_ITEMS - (_SUM_STEPS - 1) * _SUM_LANES


def _tc_sum_and_detile(tt):
    def body(t_ref, o_sum, o_rm):
        i = pl.program_id(0)

        @pl.when(i == 0)
        def _():
            o_sum[...] = jnp.zeros_like(o_sum)

        x = t_ref[...]
        valid = jnp.where(i == _SUM_STEPS - 1, _SUM_VALID_LAST, _SUM_LANES)
        lane = lax.broadcasted_iota(jnp.int32, (DIM, _SUM_LANES), 1)
        blk = jnp.where(lane < valid, x, 0.0)
        ones = jnp.ones((_SUM_LANES, 1), jnp.float32)
        part = lax.dot_general(blk, ones, (((1,), (0,)), ((), ())),
                               preferred_element_type=jnp.float32)
        o_sum[...] += part

        o_rm[0] = jnp.concatenate(
            [x[:, :_SUM_LANES // 2].T, x[:, _SUM_LANES // 2:].T], axis=1)

    return pl.pallas_call(
        body,
        grid=(_SUM_STEPS,),
        in_specs=[pl.BlockSpec((DIM, _SUM_LANES), lambda i: (0, i))],
        out_specs=[
            pl.BlockSpec((DIM, 1), lambda i: (0, 0)),
            pl.BlockSpec((1, _SUM_LANES // 2, 2 * DIM), lambda i: (i, 0, 0)),
        ],
        out_shape=[
            jax.ShapeDtypeStruct((DIM, 1), jnp.float32),
            jax.ShapeDtypeStruct((_SUM_STEPS, _SUM_LANES // 2, 2 * DIM),
                                 jnp.float32),
        ],
    )(tt)


def _erfc(x):
    t = 1.0 / (1.0 + 0.5 * x)
    poly = 0.17087277
    for c in (-0.82215223, 1.48851587, -1.13520398, 0.27886807, -0.18628806,
              0.09678418, 0.37409196, 1.00002368, -1.26551223):
        poly = c + t * poly
    return t * jnp.exp(-x * x + poly)


_PAIR_STEPS = 8
_PG = (BATCH // 128) // _PAIR_STEPS


def _pair_body(rows_ref, unc_ref, y_ref, tsum_ref, o_ref):
    def sumsq(x):
        return jnp.sum(x * x, axis=-1)

    j = rows_ref[0]
    e1 = rows_ref[1]
    e2 = rows_ref[2]

    def scale(r):
        n = jnp.sqrt(sumsq(r))
        return jnp.minimum(1.0, MAX_NORM / (n + 1e-7))[..., None]

    jn = j * scale(j)
    e1n = e1 * scale(e1)
    e2n = e2 * scale(e2)
    d1 = jnp.sqrt(sumsq(jn - e1n + 1e-6))
    d2 = jnp.sqrt(sumsq(jn - e2n + 1e-6))

    jv = jnp.exp(unc_ref[0]) + 1e-8
    v1 = jnp.exp(unc_ref[1]) + 1e-8
    v2 = jnp.exp(unc_ref[2]) + 1e-8
    s1v = jnp.sqrt(jv + v1 + 1e-8)
    s2v = jnp.sqrt(jv + v2 + 1e-8)

    p_hat = 1.0 / (1.0 + jnp.exp(d1 - d2))
    sigma = jnp.sqrt(p_hat * (1.0 - p_hat) * jnp.sqrt(s1v + s2v + 1e-8))
    z = p_hat / sigma
    w = z * 0.7071067811865476
    p = 0.5 * (2.0 - _erfc(w))
    p = jnp.clip(p, 1e-8, 1.0 - 1e-8)
    y = y_ref[...]
    bce = -(y * jnp.log(p) + (1.0 - y) * jnp.log(1.0 - p))

    i = pl.program_id(0)

    @pl.when(i == 0)
    def _():
        o_ref[0, 0] = 0.0

    o_ref[0, 0] += jnp.sum(bce)

    @pl.when(i == _PAIR_STEPS - 1)
    def _():
        o_ref[0, 0] = (o_ref[0, 0] * (1.0 / BATCH)
                       + jnp.sum(tsum_ref[...]) * (1e-6 / (NUM_ITEMS * DIM)))


def _tc_pair(rows4, unc3, y2d, tsum):
    return pl.pallas_call(
        _pair_body,
        grid=(_PAIR_STEPS,),
        in_specs=[
            pl.BlockSpec((3, _PG, 128, DIM), lambda i: (0, i, 0, 0)),
            pl.BlockSpec((3, _PG, 128), lambda i: (0, i, 0)),
            pl.BlockSpec((_PG, 128), lambda i: (i, 0)),
            pl.BlockSpec((DIM, 1), lambda i: (0, 0)),
        ],
        out_specs=pl.BlockSpec((1, 1), lambda i: (0, 0), memory_space=pltpu.SMEM),
        out_shape=jax.ShapeDtypeStruct((1, 1), jnp.float32),
    )(rows4, unc3, y2d, tsum)


def kernel(table, uncertainties, pairs, comparisons):
    idx = pairs.astype(jnp.int32).T
    idx3d = idx.reshape(_NW, _CPW, _CHUNK)
    idxr3d = ((idx >> 14) * _SUM_LANES + (idx & (_SUM_LANES // 2 - 1)) * 2
              + ((idx >> 13) & 1)).reshape(_NW, _CPW, _CHUNK)
    tsum, t_rm = _tc_sum_and_detile(table.T)
    t64 = t_rm.reshape(_SUM_STEPS * _SUM_LANES, DIM)
    rows, unc_g = _sc_gather(t64, uncertainties, idxr3d, idx3d)
    loss = _tc_pair(
        rows.reshape(3, BATCH // 128, 128, DIM),
        unc_g.reshape(3, BATCH // 128, 128),
        comparisons.reshape(BATCH // 128, 128),
        tsum,
    )
    return loss.reshape(())


# speedup vs baseline: 2.5393x; 1.0338x over previous
import functools

import jax
import jax.numpy as jnp
from jax import lax
from jax.experimental import pallas as pl
from jax.experimental.pallas import tpu as pltpu
from jax.experimental.pallas import tpu_sc as plsc

NUM_ITEMS = 1000000
DIM = 64
BATCH = 16384
MAX_NORM = 10.0

_B = 3 * BATCH
_CHUNK = 128
_NCHUNKS = _B // _CHUNK
_NW = 32
_CPW = _NCHUNKS // _NW
_HALF = _CPW // 2


def _sc_gather(t64, unc, idxr3d, idx3d):
    mesh = plsc.VectorSubcoreMesh(core_axis_name="c", subcore_axis_name="s")

    @functools.partial(
        pl.kernel,
        mesh=mesh,
        compiler_params=pltpu.CompilerParams(use_tc_tiling_on_sc=False),
        out_type=[
            jax.ShapeDtypeStruct((_B, DIM), jnp.float32),
            jax.ShapeDtypeStruct((_NW, _CPW, _CHUNK), jnp.float32),
        ],
        scratch_types=[
            pltpu.VMEM((_CPW, _CHUNK), jnp.int32),
            pltpu.VMEM((_CPW, _CHUNK), jnp.int32),
            pltpu.VMEM((_CPW * _CHUNK, DIM), jnp.float32),
            pltpu.VMEM((_CPW, _CHUNK), jnp.float32),
            pltpu.SemaphoreType.DMA,
            pltpu.SemaphoreType.DMA,
        ],
    )
    def gather_kernel(t64_h, unc_h, idxr_h, idx_h, rows_o, unc_o,
                      idxr_v, idx_v, rows_v, unc_v, sem_r, sem_u):
        wid = lax.axis_index("s") * 2 + lax.axis_index("c")
        pltpu.sync_copy(idxr_h.at[wid], idxr_v)
        pltpu.sync_copy(idx_h.at[wid], idx_v)
        descs = []
        for c in range(_CPW):
            d = pltpu.make_async_copy(
                unc_h.at[idx_v.at[c]], unc_v.at[c], sem_u)
            d.start()
            descs.append(d)
        for c in range(_CPW):
            d = pltpu.make_async_copy(
                t64_h.at[idxr_v.at[c]],
                rows_v.at[pl.ds(c * _CHUNK, _CHUNK)], sem_r)
            d.start()
            descs.append(d)
        for d in descs:
            d.wait()
        pltpu.sync_copy(rows_v,
                        rows_o.at[pl.ds(wid * _CPW * _CHUNK, _CPW * _CHUNK)])
        pltpu.sync_copy(unc_v, unc_o.at[wid])

    return gather_kernel(t64, unc, idxr3d, idx3d)


_SUM_LANES = 32768
_SUM_STEPS = -(-NUM_ITEMS // _SUM_LANES)
_SUM_VALID_LAST = NUM_ITEMS - (_SUM_STEPS - 1) * _SUM_LANES


def _tc_sum_and_detile(tt):
    def body(t_ref, o_sum, o_rm):
        i = pl.program_id(0)

        @pl.when(i == 0)
        def _():
            o_sum[...] = jnp.zeros_like(o_sum)

        x = t_ref[...]
        valid = jnp.where(i == _SUM_STEPS - 1, _SUM_VALID_LAST, _SUM_LANES)
        lane = lax.broadcasted_iota(jnp.int32, (DIM, _SUM_LANES), 1)
        blk = jnp.where(lane < valid, x, 0.0)
        ones = jnp.ones((_SUM_LANES, 1), jnp.float32)
        part = lax.dot_general(blk, ones, (((1,), (0,)), ((), ())),
                               preferred_element_type=jnp.float32)
        o_sum[...] += part

        o_rm[0] = jnp.concatenate(
            [x[:, :_SUM_LANES // 2].T, x[:, _SUM_LANES // 2:].T], axis=1)

    return pl.pallas_call(
        body,
        grid=(_SUM_STEPS,),
        in_specs=[pl.BlockSpec((DIM, _SUM_LANES), lambda i: (0, i))],
        out_specs=[
            pl.BlockSpec((DIM, 1), lambda i: (0, 0)),
            pl.BlockSpec((1, _SUM_LANES // 2, 2 * DIM), lambda i: (i, 0, 0)),
        ],
        out_shape=[
            jax.ShapeDtypeStruct((DIM, 1), jnp.float32),
            jax.ShapeDtypeStruct((_SUM_STEPS, _SUM_LANES // 2, 2 * DIM),
                                 jnp.float32),
        ],
    )(tt)


def _erfc(x):
    t = 1.0 / (1.0 + 0.5 * x)
    poly = 0.17087277
    for c in (-0.82215223, 1.48851587, -1.13520398, 0.27886807, -0.18628806,
              0.09678418, 0.37409196, 1.00002368, -1.26551223):
        poly = c + t * poly
    return t * jnp.exp(-x * x + poly)


_PAIR_STEPS = 8
_PG = (BATCH // 128) // _PAIR_STEPS


def _pair_body(j_ref, a_ref, b_ref, unc_ref, y_ref, tsum_ref, o_ref):
    def sumsq(x):
        return jnp.sum(x * x, axis=-1)

    j = j_ref[...].reshape(_PG, 128, DIM)
    e1 = a_ref[...].reshape(_PG, 128, DIM)
    e2 = b_ref[...].reshape(_PG, 128, DIM)

    def scale(r):
        n = jnp.sqrt(sumsq(r))
        return jnp.minimum(1.0, MAX_NORM / (n + 1e-7))[..., None]

    jn = j * scale(j)
    e1n = e1 * scale(e1)
    e2n = e2 * scale(e2)
    d1 = jnp.sqrt(sumsq(jn - e1n + 1e-6))
    d2 = jnp.sqrt(sumsq(jn - e2n + 1e-6))

    jv = jnp.exp(unc_ref[0]) + 1e-8
    v1 = jnp.exp(unc_ref[1]) + 1e-8
    v2 = jnp.exp(unc_ref[2]) + 1e-8
    s1v = jnp.sqrt(jv + v1 + 1e-8)
    s2v = jnp.sqrt(jv + v2 + 1e-8)

    p_hat = 1.0 / (1.0 + jnp.exp(d1 - d2))
    sigma = jnp.sqrt(p_hat * (1.0 - p_hat) * jnp.sqrt(s1v + s2v + 1e-8))
    z = p_hat / sigma
    w = z * 0.7071067811865476
    p = 0.5 * (2.0 - _erfc(w))
    p = jnp.clip(p, 1e-8, 1.0 - 1e-8)
    y = y_ref[...]
    bce = -(y * jnp.log(p) + (1.0 - y) * jnp.log(1.0 - p))

    i = pl.program_id(0)

    @pl.when(i == 0)
    def _():
        o_ref[0, 0] = 0.0

    o_ref[0, 0] += jnp.sum(bce)

    @pl.when(i == _PAIR_STEPS - 1)
    def _():
        o_ref[0, 0] = (o_ref[0, 0] * (1.0 / BATCH)
                       + jnp.sum(tsum_ref[...]) * (1e-6 / (NUM_ITEMS * DIM)))


def _tc_pair(rows2d, unc3, y2d, tsum):
    rpb = _PG * 128
    return pl.pallas_call(
        _pair_body,
        grid=(_PAIR_STEPS,),
        in_specs=[
            pl.BlockSpec((rpb, DIM), lambda i: (i, 0)),
            pl.BlockSpec((rpb, DIM), lambda i: (_PAIR_STEPS + i, 0)),
            pl.BlockSpec((rpb, DIM), lambda i: (2 * _PAIR_STEPS + i, 0)),
            pl.BlockSpec((3, _PG, 128), lambda i: (0, i, 0)),
            pl.BlockSpec((_PG, 128), lambda i: (i, 0)),
            pl.BlockSpec((DIM, 1), lambda i: (0, 0)),
        ],
        out_specs=pl.BlockSpec((1, 1), lambda i: (0, 0), memory_space=pltpu.SMEM),
        out_shape=jax.ShapeDtypeStruct((1, 1), jnp.float32),
    )(rows2d, rows2d, rows2d, unc3, y2d, tsum)


def kernel(table, uncertainties, pairs, comparisons):
    idx = pairs.astype(jnp.int32).T
    idx3d = idx.reshape(_NW, _CPW, _CHUNK)
    sh = _SUM_LANES.bit_length() - 1
    idxr3d = ((idx >> sh) * _SUM_LANES + (idx & (_SUM_LANES // 2 - 1)) * 2
              + ((idx >> (sh - 1)) & 1)).reshape(_NW, _CPW, _CHUNK)
    tsum, t_rm = _tc_sum_and_detile(table.T)
    t64 = t_rm.reshape(_SUM_STEPS * _SUM_LANES, DIM)
    rows, unc_g = _sc_gather(t64, uncertainties, idxr3d, idx3d)
    loss = _tc_pair(
        rows,
        unc_g.reshape(3, BATCH // 128, 128),
        comparisons.reshape(BATCH // 128, 128),
        tsum,
    )
    return loss.reshape(())
